# baseline (speedup 1.0000x reference)
import jax
import jax.numpy as jnp
from jax.experimental import pallas as pl


def _batchnorm(z, g, b):
    mu = jnp.mean(z, axis=0)
    var = jnp.var(z, axis=0)
    return (z - mu) / jnp.sqrt(var + 1e-5) * g + b


def _masked_batchnorm(z, g, b, m, cnt):
    mu = jnp.sum(z * m, axis=0) / cnt
    var = jnp.sum(jnp.square((z - mu) * m), axis=0) / cnt
    return (z - mu) / jnp.sqrt(var + 1e-5) * g + b


def _gkmlp(e, W1, b1, W2, b2, W3, b3, g1, be1, g2, be2, m, cnt):
    h = jax.nn.elu(_masked_batchnorm(e @ W1 + b1, g1, be1, m, cnt))
    h = jax.nn.elu(_masked_batchnorm(h @ W2 + b2, g2, be2, m, cnt))
    return h @ W3 + b3


def _lower_kernel(h_ref, w_ref, b_ref, o_ref):
    o_ref[...] = h_ref[...] @ w_ref[...] + b_ref[...]


def kernel(x, edge_index, pos, y, lift_W, lift_b, lin_W, k1_W, k1_b, k2_W, k2_b, k3_W, k3_b, kbn1_g, kbn1_b, kbn2_g, kbn2_b, bn_g, bn_b, lower_W, lower_b):
    n = x.shape[0]
    keys = edge_index[0] * n + edge_index[1]
    uk = jnp.unique(keys, size=keys.shape[0], fill_value=-1)
    emask = uk >= 0
    uks = jnp.where(emask, uk, 0)
    src = uks // n
    dst = uks % n
    loops = jnp.arange(n, dtype=src.dtype)
    src = jnp.concatenate([src, loops])
    dst = jnp.concatenate([dst, loops])
    mask_f = jnp.concatenate([emask.astype(jnp.float32), jnp.ones((n,), dtype=jnp.float32)])[:, None]
    cnt = jnp.sum(mask_f)
    deg = jax.ops.segment_sum(mask_f[:, 0], src, num_segments=n)
    deg_e = deg[dst][:, None]
    h = jnp.concatenate([x, pos, y[None, :] * jnp.ones_like(pos)], axis=1) @ lift_W + lift_b
    e_feat = jnp.concatenate([pos[src], pos[dst], x[src], x[dst]], axis=1)
    L = lin_W.shape[0]
    for l in range(L):
        w = _gkmlp(e_feat, k1_W[l], k1_b[l], k2_W[l], k2_b[l], k3_W[l], k3_b[l], kbn1_g[l], kbn1_b[l], kbn2_g[l], kbn2_b[l], mask_f, cnt)
        w = w / deg_e
        w = w * mask_f
        agg = jax.ops.segment_sum(w * h[src], dst, num_segments=n)
        h = jax.nn.elu(_batchnorm(h @ lin_W[l] + agg, bn_g[l], bn_b[l]))
    out = pl.pallas_call(
        _lower_kernel,
        out_shape=jax.ShapeDtypeStruct((n, lower_W.shape[1]), h.dtype),
    )(h, lower_W, lower_b)
    return out


# speedup vs baseline: 1.5386x; 1.5386x over previous
import functools

import jax
import jax.numpy as jnp
from jax import lax
from jax.experimental import pallas as pl
from jax.experimental.pallas import tpu as pltpu
from jax.experimental.pallas import tpu_sc as plsc

N = 10000
E = 320000
F = 128
DIM = 3
H = 128
EP = 330240
LOOP_ROWS = 80
E_ROWS = 2500
EP_ROWS = 2580
NW = 32
EW = EP // NW
CK = 80
NCH = EW // CK
TCK = 512
TGRID = EP // TCK
NPT = N // 16
F32 = jnp.float32


def _elu(v):
    return jnp.where(v > 0, v, jnp.exp(v) - 1.0)


def _edge_body(k_ref, src_ref, dst_ref, msk_ref, cnt_ref):
    k = k_ref[...]
    col0 = jnp.concatenate([k[0:1, 127:128], k[0:E_ROWS - 1, 127:128]], axis=0)
    prev = jnp.concatenate([col0, k[:, 0:127]], axis=1)
    r = lax.broadcasted_iota(jnp.int32, (E_ROWS, 128), 0)
    c = lax.broadcasted_iota(jnp.int32, (E_ROWS, 128), 1)
    first = jnp.logical_and(r == 0, c == 0)
    me = jnp.logical_or(k != prev, first).astype(F32)
    srcE = k // N
    dstE = k - srcE * N
    li = (lax.broadcasted_iota(jnp.int32, (LOOP_ROWS, 128), 0) * 128
          + lax.broadcasted_iota(jnp.int32, (LOOP_ROWS, 128), 1))
    lvalid = li < N
    lv = jnp.where(lvalid, li, 0)
    src_ref[...] = jnp.concatenate([srcE, lv], axis=0)
    dst_ref[...] = jnp.concatenate([dstE, lv], axis=0)
    mf = jnp.concatenate([me, lvalid.astype(F32)], axis=0)
    msk_ref[...] = mf
    cnt_ref[...] = jnp.sum(mf).reshape(1, 1)


def _edge_call(skeys2d):
    return pl.pallas_call(
        _edge_body,
        out_shape=[
            jax.ShapeDtypeStruct((EP_ROWS, 128), jnp.int32),
            jax.ShapeDtypeStruct((EP_ROWS, 128), jnp.int32),
            jax.ShapeDtypeStruct((EP_ROWS, 128), F32),
            jax.ShapeDtypeStruct((1, 1), F32),
        ],
    )(skeys2d)


def _deg_body(src_hbm, msk_hbm, out_hbm, aggsh, sidx, mv, selbuf, onesbuf, zbuf):
    c = lax.axis_index("c")
    s = lax.axis_index("s")
    wid = s * 2 + c
    base = wid * EW

    def zfill(i, _):
        for v in range(8):
            zbuf[i, pl.ds(v * 16, 16)] = jnp.zeros((16,), F32)
        return 0

    lax.fori_loop(0, 128, zfill, 0)

    def ofill(i, _):
        for v in range(8):
            onesbuf[i, pl.ds(v * 16, 16)] = jnp.ones((16,), F32)
        return 0

    lax.fori_loop(0, CK, ofill, 0)
    for t in range(5):
        pltpu.sync_copy(zbuf, aggsh.at[pl.ds(s * 640 + t * 128, 128)])
    plsc.subcore_barrier()

    def chunk(j, _):
        off = base + j * CK
        pltpu.sync_copy(src_hbm.at[pl.ds(off, CK)], sidx)
        pltpu.sync_copy(msk_hbm.at[pl.ds(off, CK)], mv)
        for k in range(CK // 16):
            sl = pl.ds(k * 16, 16)
            i16 = sidx[sl]
            m16 = mv[sl]
            selbuf[sl] = jnp.where(m16 > 0.5, i16, N)
        pltpu.sync_copy(onesbuf, aggsh.at[selbuf], add=True)
        return 0

    lax.fori_loop(0, NCH, chunk, 0)
    plsc.subcore_barrier()
    for t in range(5):
        rows = pl.ds(s * 640 + t * 128, 128)
        pltpu.sync_copy(aggsh.at[rows], out_hbm.at[c, rows])


def _deg_call(src, msk):
    fn = functools.partial(
        pl.kernel,
        mesh=plsc.VectorSubcoreMesh(core_axis_name="c", subcore_axis_name="s"),
        out_type=jax.ShapeDtypeStruct((2, 10240, H), F32),
        scratch_types=[
            pltpu.VMEM_SHARED((10240, H), F32),
            pltpu.VMEM((CK,), jnp.int32),
            pltpu.VMEM((CK,), F32),
            pltpu.VMEM((CK,), jnp.int32),
            pltpu.VMEM((CK, H), F32),
            pltpu.VMEM((128, H), F32),
        ],
    )(_deg_body)
    return fn(src, msk)


def _gather_body(A_hbm, B_hbm, src_hbm, dst_hbm, Ag_hbm, Bg_hbm,
                 sidx, didx, bufA, bufB, semA, semB):
    c = lax.axis_index("c")
    s = lax.axis_index("s")
    wid = s * 2 + c
    base = wid * EW

    def chunk(j, _):
        off = base + j * CK
        pltpu.sync_copy(src_hbm.at[pl.ds(off, CK)], sidx)
        pltpu.sync_copy(dst_hbm.at[pl.ds(off, CK)], didx)
        cpA = pltpu.async_copy(A_hbm.at[sidx], bufA, semA)
        cpB = pltpu.async_copy(B_hbm.at[didx], bufB, semB)
        cpA.wait()
        cpB.wait()
        pltpu.sync_copy(bufA, Ag_hbm.at[pl.ds(off, CK)])
        pltpu.sync_copy(bufB, Bg_hbm.at[pl.ds(off, CK)])
        return 0

    lax.fori_loop(0, NCH, chunk, 0)


def _gather_call(A, B, src, dst):
    fn = functools.partial(
        pl.kernel,
        mesh=plsc.VectorSubcoreMesh(core_axis_name="c", subcore_axis_name="s"),
        out_type=[
            jax.ShapeDtypeStruct((EP, H), F32),
            jax.ShapeDtypeStruct((EP, H), F32),
        ],
        scratch_types=[
            pltpu.VMEM((CK,), jnp.int32),
            pltpu.VMEM((CK,), jnp.int32),
            pltpu.VMEM((CK, H), F32),
            pltpu.VMEM((CK, H), F32),
            pltpu.SemaphoreType.DMA,
            pltpu.SemaphoreType.DMA,
        ],
    )(_gather_body)
    return fn(A, B, src, dst)


def _msg_body(w_hbm, h_hbm, src_hbm, dst_hbm, out_hbm,
              aggsh, sidx, didx, wbuf, hbuf, zbuf, semw, semh):
    c = lax.axis_index("c")
    s = lax.axis_index("s")
    wid = s * 2 + c
    base = wid * EW

    def zero(i, _):
        for v in range(8):
            zbuf[i, pl.ds(v * 16, 16)] = jnp.zeros((16,), F32)
        return 0

    lax.fori_loop(0, 128, zero, 0)
    for t in range(5):
        pltpu.sync_copy(zbuf, aggsh.at[pl.ds(s * 640 + t * 128, 128)])
    plsc.subcore_barrier()

    def chunk(j, _):
        off = base + j * CK
        pltpu.sync_copy(src_hbm.at[pl.ds(off, CK)], sidx)
        pltpu.sync_copy(dst_hbm.at[pl.ds(off, CK)], didx)
        cph = pltpu.async_copy(h_hbm.at[sidx], hbuf, semh)
        cpw = pltpu.async_copy(w_hbm.at[pl.ds(off, CK)], wbuf, semw)
        cph.wait()
        cpw.wait()

        def mul(e, _):
            for v in range(8):
                sl = pl.ds(v * 16, 16)
                wbuf[e, sl] = wbuf[e, sl] * hbuf[e, sl]
            return 0

        lax.fori_loop(0, CK, mul, 0)
        pltpu.sync_copy(wbuf, aggsh.at[didx], add=True)
        return 0

    lax.fori_loop(0, NCH, chunk, 0)
    plsc.subcore_barrier()
    for t in range(5):
        rows = pl.ds(s * 640 + t * 128, 128)
        pltpu.sync_copy(aggsh.at[rows], out_hbm.at[c, rows])


def _msg_call(w, h, src, dst):
    fn = functools.partial(
        pl.kernel,
        mesh=plsc.VectorSubcoreMesh(core_axis_name="c", subcore_axis_name="s"),
        out_type=jax.ShapeDtypeStruct((2, 10240, H), F32),
        scratch_types=[
            pltpu.VMEM_SHARED((10240, H), F32),
            pltpu.VMEM((CK,), jnp.int32),
            pltpu.VMEM((CK,), jnp.int32),
            pltpu.VMEM((CK, H), F32),
            pltpu.VMEM((CK, H), F32),
            pltpu.VMEM((128, H), F32),
            pltpu.SemaphoreType.DMA,
            pltpu.SemaphoreType.DMA,
        ],
    )(_msg_body)
    return fn(w, h, src, dst)


def _node_body(x_ref, p_ref, y_ref, wx_ref, wp_ref, wy_ref, lb_ref, h_ref):
    yrow = y_ref[...] @ wy_ref[...] + lb_ref[...]
    h_ref[...] = x_ref[...] @ wx_ref[...] + p_ref[...] @ wp_ref[...] + yrow


def _node_call(x, pos128, y128, Wx, Wp, Wy, lb):
    return pl.pallas_call(
        _node_body,
        out_shape=jax.ShapeDtypeStruct((N, H), F32),
    )(x, pos128, y128, Wx, Wp, Wy, lb)


def _ab_body(x_ref, p_ref, wxs_ref, wps_ref, wxd_ref, wpd_ref, a_ref, b_ref):
    a_ref[0] = x_ref[...] @ wxs_ref[0] + p_ref[...] @ wps_ref[0]
    b_ref[0] = x_ref[...] @ wxd_ref[0] + p_ref[...] @ wpd_ref[0]


def _ab_call(x, pos128, Wxs, Wps, Wxd, Wpd, L):
    full = lambda i: (0, 0)
    wspec = pl.BlockSpec((1, H, H), lambda i: (i, 0, 0))
    return pl.pallas_call(
        _ab_body,
        grid=(L,),
        in_specs=[
            pl.BlockSpec((N, H), full),
            pl.BlockSpec((N, H), full),
            wspec, wspec, wspec, wspec,
        ],
        out_specs=[
            pl.BlockSpec((1, N, H), lambda i: (i, 0, 0)),
            pl.BlockSpec((1, N, H), lambda i: (i, 0, 0)),
        ],
        out_shape=[
            jax.ShapeDtypeStruct((L, N, H), F32),
            jax.ShapeDtypeStruct((L, N, H), F32),
        ],
    )(x, pos128, Wxs, Wps, Wxd, Wpd)


def _stats1_body(ag_ref, bg_ref, m_ref, o_ref):
    i = pl.program_id(0)

    @pl.when(i == 0)
    def _():
        o_ref[...] = jnp.zeros((2, 128), F32)

    z = ag_ref[...] + bg_ref[...]
    zm = z * m_ref[...]
    o_ref[0:1, :] += jnp.sum(zm, axis=0, keepdims=True)
    o_ref[1:2, :] += jnp.sum(z * zm, axis=0, keepdims=True)


def _stats1_call(Ag, Bg, msk1):
    blk = pl.BlockSpec((TCK, H), lambda i: (i, 0))
    mblk = pl.BlockSpec((TCK, 1), lambda i: (i, 0))
    return pl.pallas_call(
        _stats1_body,
        grid=(TGRID,),
        in_specs=[blk, blk, mblk],
        out_specs=pl.BlockSpec((2, 128), lambda i: (0, 0)),
        out_shape=jax.ShapeDtypeStruct((2, 128), F32),
    )(Ag, Bg, msk1)


def _pass2_body(ag_ref, bg_ref, m_ref, mu_ref, rg_ref, be_ref, w2_ref,
                z2_ref, o_ref):
    i = pl.program_id(0)

    @pl.when(i == 0)
    def _():
        o_ref[...] = jnp.zeros((2, 128), F32)

    z1 = ag_ref[...] + bg_ref[...]
    a1 = _elu((z1 - mu_ref[...]) * rg_ref[...] + be_ref[...])
    z2 = jnp.dot(a1, w2_ref[...], preferred_element_type=F32)
    z2_ref[...] = z2
    zm = z2 * m_ref[...]
    o_ref[0:1, :] += jnp.sum(zm, axis=0, keepdims=True)
    o_ref[1:2, :] += jnp.sum(z2 * zm, axis=0, keepdims=True)


def _pass2_call(Ag, Bg, msk1, mu1, r1g, be1, W2):
    blk = pl.BlockSpec((TCK, H), lambda i: (i, 0))
    mblk = pl.BlockSpec((TCK, 1), lambda i: (i, 0))
    row = pl.BlockSpec((1, 128), lambda i: (0, 0))
    return pl.pallas_call(
        _pass2_body,
        grid=(TGRID,),
        in_specs=[blk, blk, mblk, row, row, row,
                  pl.BlockSpec((H, H), lambda i: (0, 0))],
        out_specs=[blk, pl.BlockSpec((2, 128), lambda i: (0, 0))],
        out_shape=[
            jax.ShapeDtypeStruct((EP, H), F32),
            jax.ShapeDtypeStruct((2, 128), F32),
        ],
    )(Ag, Bg, msk1, mu1, r1g, be1, W2)


def _pass3_body(z2_ref, sc_ref, mu_ref, rg_ref, be_ref, w3_ref, b3_ref, o_ref):
    a2 = _elu((z2_ref[...] - mu_ref[...]) * rg_ref[...] + be_ref[...])
    w = jnp.dot(a2, w3_ref[...], preferred_element_type=F32) + b3_ref[...]
    o_ref[...] = w * sc_ref[...]


def _pass3_call(z2, scale1, mu2, r2g, be2, W3, b3):
    blk = pl.BlockSpec((TCK, H), lambda i: (i, 0))
    mblk = pl.BlockSpec((TCK, 1), lambda i: (i, 0))
    row = pl.BlockSpec((1, 128), lambda i: (0, 0))
    return pl.pallas_call(
        _pass3_body,
        grid=(TGRID,),
        in_specs=[blk, mblk, row, row, row,
                  pl.BlockSpec((H, H), lambda i: (0, 0)), row],
        out_specs=blk,
        out_shape=jax.ShapeDtypeStruct((EP, H), F32),
    )(z2, scale1, mu2, r2g, be2, W3, b3)


def _update_body(h_ref, a0_ref, a1_ref, d0_ref, d1_ref, lw_ref, g_ref, b_ref,
                 o_ref):
    zn = (jnp.dot(h_ref[...], lw_ref[...], preferred_element_type=F32)
          + (a0_ref[...] + a1_ref[...]) / (d0_ref[...] + d1_ref[...]))
    mu = jnp.mean(zn, axis=0, keepdims=True)
    var = jnp.mean(jnp.square(zn - mu), axis=0, keepdims=True)
    o_ref[...] = _elu((zn - mu) * lax.rsqrt(var + 1e-5) * g_ref[...] + b_ref[...])


def _update_call(h, agg0, agg1, d0, d1, lw, g, b):
    return pl.pallas_call(
        _update_body,
        out_shape=jax.ShapeDtypeStruct((N, H), F32),
    )(h, agg0, agg1, d0, d1, lw, g, b)


def _final_body(h_ref, w_ref, b_ref, o_ref):
    o_ref[...] = jnp.dot(h_ref[...], w_ref[...], preferred_element_type=F32) + b_ref[...]


def _final_call(h, wpad, bpad):
    return pl.pallas_call(
        _final_body,
        out_shape=jax.ShapeDtypeStruct((N, 128), F32),
    )(h, wpad, bpad)


def kernel(x, edge_index, pos, y, lift_W, lift_b, lin_W, k1_W, k1_b, k2_W,
           k2_b, k3_W, k3_b, kbn1_g, kbn1_b, kbn2_g, kbn2_b, bn_g, bn_b,
           lower_W, lower_b):
    L = lin_W.shape[0]
    keys = edge_index[0] * N + edge_index[1]
    skeys2d = jnp.sort(keys).reshape(E_ROWS, 128)

    src2d, dst2d, msk2d, cnt11 = _edge_call(skeys2d)
    src = src2d.reshape(EP)
    dst = dst2d.reshape(EP)
    msk = msk2d.reshape(EP)
    msk1 = msk2d.reshape(EP, 1)
    cnt = cnt11[0, 0]

    degt = _deg_call(src, msk)
    d0 = degt[0, :N]
    d1 = degt[1, :N]

    pos128 = jnp.pad(pos, ((0, 0), (0, 128 - DIM)))
    y128 = jnp.pad(y, (0, 128 - DIM)).reshape(1, 128)
    pad_w = lambda w: jnp.pad(w, ((0, 128 - w.shape[0]), (0, 0)))
    Wx = lift_W[:F]
    Wp = pad_w(lift_W[F:F + DIM])
    Wy = pad_w(lift_W[F + DIM:F + 2 * DIM])
    lb = lift_b.reshape(1, 128)

    h = _node_call(x, pos128, y128, Wx, Wp, Wy, lb)

    pad_w3 = lambda w: jnp.pad(w, ((0, 0), (0, 128 - w.shape[1]), (0, 0)))
    Wps = pad_w3(k1_W[:, 0:DIM, :])
    Wpd = pad_w3(k1_W[:, DIM:2 * DIM, :])
    Wxs = k1_W[:, 2 * DIM:2 * DIM + F, :]
    Wxd = k1_W[:, 2 * DIM + F:2 * DIM + 2 * F, :]
    A, B = _ab_call(x, pos128, Wxs, Wps, Wxd, Wpd, L)

    for l in range(L):
        Ag, Bg = _gather_call(A[l], B[l], src, dst)
        st1 = _stats1_call(Ag, Bg, msk1)
        mu1 = (st1[0:1] / cnt)
        var1 = st1[1:2] / cnt - mu1 * mu1
        r1g = lax.rsqrt(var1 + 1e-5) * kbn1_g[l].reshape(1, 128)
        be1 = kbn1_b[l].reshape(1, 128)
        z2, st2 = _pass2_call(Ag, Bg, msk1, mu1, r1g, be1, k2_W[l])
        mu2 = st2[0:1] / cnt
        var2 = st2[1:2] / cnt - mu2 * mu2
        r2g = lax.rsqrt(var2 + 1e-5) * kbn2_g[l].reshape(1, 128)
        be2 = kbn2_b[l].reshape(1, 128)
        w = _pass3_call(z2, msk1, mu2, r2g, be2, k3_W[l],
                        k3_b[l].reshape(1, 128))
        agg2 = _msg_call(w, h, src, dst)
        h = _update_call(h, agg2[0, :N], agg2[1, :N], d0, d1, lin_W[l],
                         bn_g[l].reshape(1, 128), bn_b[l].reshape(1, 128))

    wpad = jnp.pad(lower_W, ((0, 0), (0, 128 - lower_W.shape[1])))
    bpad = jnp.pad(lower_b, (0, 128 - lower_b.shape[0])).reshape(1, 128)
    out128 = _final_call(h, wpad, bpad)
    return out128[:, :lower_W.shape[1]]


# speedup vs baseline: 1.8203x; 1.1831x over previous
import functools

import jax
import jax.numpy as jnp
from jax import lax
from jax.experimental import pallas as pl
from jax.experimental.pallas import tpu as pltpu
from jax.experimental.pallas import tpu_sc as plsc

N = 10000
E = 320000
F = 128
DIM = 3
H = 128
EP = 331776
LOOP_ROWS = 92
E_ROWS = 2500
EP_ROWS = 2592
NW = 32
EW = EP // NW
CK = 128
NCH = EW // CK
TCK = 512
TGRID = EP // TCK
NPT = N // 16
F32 = jnp.float32


def _elu(v):
    return jnp.where(v > 0, v, jnp.exp(v) - 1.0)


def _edge_body(k_ref, src_ref, dst_ref, msk_ref, cnt_ref):
    k = k_ref[...]
    col0 = jnp.concatenate([k[0:1, 127:128], k[0:E_ROWS - 1, 127:128]], axis=0)
    prev = jnp.concatenate([col0, k[:, 0:127]], axis=1)
    r = lax.broadcasted_iota(jnp.int32, (E_ROWS, 128), 0)
    c = lax.broadcasted_iota(jnp.int32, (E_ROWS, 128), 1)
    first = jnp.logical_and(r == 0, c == 0)
    me = jnp.logical_or(k != prev, first).astype(F32)
    srcE = k // N
    dstE = k - srcE * N
    li = (lax.broadcasted_iota(jnp.int32, (LOOP_ROWS, 128), 0) * 128
          + lax.broadcasted_iota(jnp.int32, (LOOP_ROWS, 128), 1))
    lvalid = li < N
    lv = jnp.where(lvalid, li, 0)
    src_ref[...] = jnp.concatenate([srcE, lv], axis=0)
    dst_ref[...] = jnp.concatenate([dstE, lv], axis=0)
    mf = jnp.concatenate([me, lvalid.astype(F32)], axis=0)
    msk_ref[...] = mf
    cnt_ref[...] = jnp.sum(mf).reshape(1, 1)


def _edge_call(skeys2d):
    return pl.pallas_call(
        _edge_body,
        out_shape=[
            jax.ShapeDtypeStruct((EP_ROWS, 128), jnp.int32),
            jax.ShapeDtypeStruct((EP_ROWS, 128), jnp.int32),
            jax.ShapeDtypeStruct((EP_ROWS, 128), F32),
            jax.ShapeDtypeStruct((1, 1), F32),
        ],
    )(skeys2d)


def _deg_body(src_hbm, msk_hbm, out_hbm, aggsh, sidx2, mv2, selbuf, onesbuf):
    c = lax.axis_index("c")
    s = lax.axis_index("s")
    wid = s * 2 + c

    def ofill(i, _):
        for v in range(8):
            onesbuf[i, pl.ds(v * 16, 16)] = jnp.ones((16,), F32)
        return 0

    lax.fori_loop(0, CK, ofill, 0)
    pltpu.sync_copy(src_hbm.at[wid], sidx2)
    pltpu.sync_copy(msk_hbm.at[wid], mv2)

    def zfill(i, _):
        for v in range(8):
            onesbuf[i, pl.ds(v * 16, 16)] = jnp.zeros((16,), F32)
        return 0

    lax.fori_loop(0, 128, zfill, 0)
    for t in range(5):
        pltpu.sync_copy(onesbuf, aggsh.at[pl.ds(s * 640 + t * 128, 128)])
    lax.fori_loop(0, CK, ofill, 0)
    plsc.subcore_barrier()

    def chunk(j, _):
        for k in range(CK // 16):
            sl = pl.ds(k * 16, 16)
            selbuf[sl] = jnp.where(mv2[j, sl] > 0.5, sidx2[j, sl], N)
        pltpu.sync_copy(onesbuf, aggsh.at[selbuf], add=True)
        return 0

    lax.fori_loop(0, NCH, chunk, 0)
    plsc.subcore_barrier()
    for t in range(5):
        rows = pl.ds(s * 640 + t * 128, 128)
        pltpu.sync_copy(aggsh.at[rows], out_hbm.at[c, rows])


def _deg_call(src3, msk3):
    fn = functools.partial(
        pl.kernel,
        mesh=plsc.VectorSubcoreMesh(core_axis_name="c", subcore_axis_name="s"),
        out_type=jax.ShapeDtypeStruct((2, 10240, H), F32),
        scratch_types=[
            pltpu.VMEM_SHARED((10240, H), F32),
            pltpu.VMEM((NCH, CK), jnp.int32),
            pltpu.VMEM((NCH, CK), F32),
            pltpu.VMEM((CK,), jnp.int32),
            pltpu.VMEM((CK, H), F32),
        ],
    )(_deg_body)
    return fn(src3, msk3)


def _gather_body(A_hbm, B_hbm, src_hbm, dst_hbm, Z_hbm,
                 sidx2, didx2, bufA, bufB, semA, semB, semW):
    c = lax.axis_index("c")
    s = lax.axis_index("s")
    wid = s * 2 + c
    base = wid * EW
    pltpu.sync_copy(src_hbm.at[wid], sidx2)
    pltpu.sync_copy(dst_hbm.at[wid], didx2)

    def fire(j, p):
        pltpu.async_copy(A_hbm.at[sidx2.at[j]], bufA.at[p], semA.at[p])
        pltpu.async_copy(B_hbm.at[didx2.at[j]], bufB.at[p], semB.at[p])

    fire(0, 0)

    def chunk(j, _):
        p = lax.rem(j, 2)

        @pl.when(j >= 1)
        def _():
            off1 = base + (j - 1) * CK
            pltpu.make_async_copy(
                bufA.at[1 - p], Z_hbm.at[pl.ds(off1, CK)], semW.at[1 - p]).wait()

        @pl.when(j + 1 < NCH)
        def _():
            fire(j + 1, 1 - p)

        pltpu.make_async_copy(A_hbm.at[sidx2.at[j]], bufA.at[p], semA.at[p]).wait()
        pltpu.make_async_copy(B_hbm.at[didx2.at[j]], bufB.at[p], semB.at[p]).wait()

        def add(e, _):
            for v in range(8):
                sl = pl.ds(v * 16, 16)
                bufA[p, e, sl] = bufA[p, e, sl] + bufB[p, e, sl]
            return 0

        lax.fori_loop(0, CK, add, 0, unroll=4)
        off = base + j * CK
        pltpu.async_copy(bufA.at[p], Z_hbm.at[pl.ds(off, CK)], semW.at[p])
        return 0

    lax.fori_loop(0, NCH, chunk, 0)
    j = NCH - 1
    p = j % 2
    off = base + j * CK
    pltpu.make_async_copy(bufA.at[p], Z_hbm.at[pl.ds(off, CK)], semW.at[p]).wait()


def _gather_call(A, B, src3, dst3):
    fn = functools.partial(
        pl.kernel,
        mesh=plsc.VectorSubcoreMesh(core_axis_name="c", subcore_axis_name="s"),
        out_type=jax.ShapeDtypeStruct((EP, H), F32),
        scratch_types=[
            pltpu.VMEM((NCH, CK), jnp.int32),
            pltpu.VMEM((NCH, CK), jnp.int32),
            pltpu.VMEM((2, CK, H), F32),
            pltpu.VMEM((2, CK, H), F32),
            pltpu.SemaphoreType.DMA((2,)),
            pltpu.SemaphoreType.DMA((2,)),
            pltpu.SemaphoreType.DMA((2,)),
        ],
    )(_gather_body)
    return fn(A, B, src3, dst3)


CKM = 64
NCHM = EW // CKM
MB = 27
NMB = NCHM // MB


def _msg_body(w_hbm, h_hbm, src_hbm, dst_hbm, out_hbm,
              aggsh, sidxb, didxv, wbuf, hbuf, semw, semh, sema):
    c = lax.axis_index("c")
    s = lax.axis_index("s")
    wid = s * 2 + c
    base = wid * EW

    def zero(i, _):
        for v in range(8):
            wbuf[0, i, pl.ds(v * 16, 16)] = jnp.zeros((16,), F32)
        return 0

    lax.fori_loop(0, CKM, zero, 0)
    for t in range(10):
        pltpu.sync_copy(wbuf.at[0], aggsh.at[pl.ds(s * 640 + t * CKM, CKM)])
    plsc.subcore_barrier()

    def fire(jj, row, p):
        off = base + jj * CKM
        pltpu.async_copy(h_hbm.at[sidxb.at[pl.ds(row * CKM, CKM)]],
                         hbuf.at[p], semh.at[p])
        pltpu.async_copy(w_hbm.at[pl.ds(off, CKM)], wbuf.at[p], semw.at[p])

    def wait_w(p):
        pltpu.make_async_copy(wbuf.at[p], aggsh.at[didxv.at[0]], sema.at[p]).wait()

    for b in range(NMB):
        j0 = b * MB

        if b > 0:
            wait_w((j0 - 1) % 2)
        pltpu.sync_copy(src_hbm.at[pl.ds(base + j0 * CKM, MB * CKM)], sidxb)
        fire(j0, 0, j0 % 2)

        def inner(j2, _):
            jj = j0 + j2
            p = lax.rem(jj, 2)

            @pl.when(j2 >= 1)
            def _():
                wait_w(1 - p)

            @pl.when(j2 + 1 < MB)
            def _():
                fire(jj + 1, j2 + 1, 1 - p)

            off = base + jj * CKM
            pltpu.make_async_copy(
                h_hbm.at[sidxb.at[pl.ds(j2 * CKM, CKM)]],
                hbuf.at[p], semh.at[p]).wait()
            pltpu.make_async_copy(
                w_hbm.at[pl.ds(off, CKM)], wbuf.at[p], semw.at[p]).wait()

            def mul(e, _):
                for v in range(8):
                    sl = pl.ds(v * 16, 16)
                    wbuf[p, e, sl] = wbuf[p, e, sl] * hbuf[p, e, sl]
                return 0

            lax.fori_loop(0, CKM, mul, 0, unroll=4)
            pltpu.sync_copy(dst_hbm.at[pl.ds(off, CKM)], didxv.at[p])
            pltpu.async_copy(wbuf.at[p], aggsh.at[didxv.at[p]], sema.at[p],
                             add=True)
            return 0

        lax.fori_loop(0, MB, inner, 0)

    wait_w((NCHM - 1) % 2)
    plsc.subcore_barrier()
    for t in range(5):
        rows = pl.ds(s * 640 + t * 128, 128)
        pltpu.sync_copy(aggsh.at[rows], out_hbm.at[c, rows])


def _msg_call(w, h, srcM, dstM):
    fn = functools.partial(
        pl.kernel,
        mesh=plsc.VectorSubcoreMesh(core_axis_name="c", subcore_axis_name="s"),
        out_type=jax.ShapeDtypeStruct((2, 10240, H), F32),
        scratch_types=[
            pltpu.VMEM_SHARED((10240, H), F32),
            pltpu.VMEM((MB * CKM,), jnp.int32),
            pltpu.VMEM((2, CKM), jnp.int32),
            pltpu.VMEM((2, CKM, H), F32),
            pltpu.VMEM((2, CKM, H), F32),
            pltpu.SemaphoreType.DMA((2,)),
            pltpu.SemaphoreType.DMA((2,)),
            pltpu.SemaphoreType.DMA((2,)),
        ],
    )(_msg_body)
    return fn(w, h, srcM, dstM)


def _node_body(x_ref, p_ref, y_ref, wx_ref, wp_ref, wy_ref, lb_ref, h_ref):
    yrow = y_ref[...] @ wy_ref[...] + lb_ref[...]
    h_ref[...] = x_ref[...] @ wx_ref[...] + p_ref[...] @ wp_ref[...] + yrow


def _node_call(x, pos128, y128, Wx, Wp, Wy, lb):
    return pl.pallas_call(
        _node_body,
        out_shape=jax.ShapeDtypeStruct((N, H), F32),
    )(x, pos128, y128, Wx, Wp, Wy, lb)


def _ab_body(x_ref, p_ref, wxs_ref, wps_ref, wxd_ref, wpd_ref, a_ref, b_ref):
    a_ref[0] = x_ref[...] @ wxs_ref[0] + p_ref[...] @ wps_ref[0]
    b_ref[0] = x_ref[...] @ wxd_ref[0] + p_ref[...] @ wpd_ref[0]


def _ab_call(x, pos128, Wxs, Wps, Wxd, Wpd, L):
    full = lambda i: (0, 0)
    wspec = pl.BlockSpec((1, H, H), lambda i: (i, 0, 0))
    return pl.pallas_call(
        _ab_body,
        grid=(L,),
        in_specs=[
            pl.BlockSpec((N, H), full),
            pl.BlockSpec((N, H), full),
            wspec, wspec, wspec, wspec,
        ],
        out_specs=[
            pl.BlockSpec((1, N, H), lambda i: (i, 0, 0)),
            pl.BlockSpec((1, N, H), lambda i: (i, 0, 0)),
        ],
        out_shape=[
            jax.ShapeDtypeStruct((L, N, H), F32),
            jax.ShapeDtypeStruct((L, N, H), F32),
        ],
    )(x, pos128, Wxs, Wps, Wxd, Wpd)


def _stats1_body(z_ref, m_ref, o_ref):
    i = pl.program_id(0)

    @pl.when(i == 0)
    def _():
        o_ref[...] = jnp.zeros((2, 128), F32)

    z = z_ref[...]
    zm = z * m_ref[...]
    o_ref[0:1, :] += jnp.sum(zm, axis=0, keepdims=True)
    o_ref[1:2, :] += jnp.sum(z * zm, axis=0, keepdims=True)


def _stats1_call(z1, msk1):
    blk = pl.BlockSpec((TCK, H), lambda i: (i, 0))
    mblk = pl.BlockSpec((TCK, 1), lambda i: (i, 0))
    return pl.pallas_call(
        _stats1_body,
        grid=(TGRID,),
        in_specs=[blk, mblk],
        out_specs=pl.BlockSpec((2, 128), lambda i: (0, 0)),
        out_shape=jax.ShapeDtypeStruct((2, 128), F32),
    )(z1, msk1)


def _pass2_body(z1_ref, m_ref, mu_ref, rg_ref, be_ref, w2_ref,
                z2_ref, o_ref):
    i = pl.program_id(0)

    @pl.when(i == 0)
    def _():
        o_ref[...] = jnp.zeros((2, 128), F32)

    a1 = _elu((z1_ref[...] - mu_ref[...]) * rg_ref[...] + be_ref[...])
    z2 = jnp.dot(a1, w2_ref[...], preferred_element_type=F32)
    z2_ref[...] = z2
    zm = z2 * m_ref[...]
    o_ref[0:1, :] += jnp.sum(zm, axis=0, keepdims=True)
    o_ref[1:2, :] += jnp.sum(z2 * zm, axis=0, keepdims=True)


def _pass2_call(z1, msk1, mu1, r1g, be1, W2):
    blk = pl.BlockSpec((TCK, H), lambda i: (i, 0))
    mblk = pl.BlockSpec((TCK, 1), lambda i: (i, 0))
    row = pl.BlockSpec((1, 128), lambda i: (0, 0))
    return pl.pallas_call(
        _pass2_body,
        grid=(TGRID,),
        in_specs=[blk, mblk, row, row, row,
                  pl.BlockSpec((H, H), lambda i: (0, 0))],
        out_specs=[blk, pl.BlockSpec((2, 128), lambda i: (0, 0))],
        out_shape=[
            jax.ShapeDtypeStruct((EP, H), F32),
            jax.ShapeDtypeStruct((2, 128), F32),
        ],
    )(z1, msk1, mu1, r1g, be1, W2)


def _pass3_body(z2_ref, sc_ref, mu_ref, rg_ref, be_ref, w3_ref, b3_ref, o_ref):
    a2 = _elu((z2_ref[...] - mu_ref[...]) * rg_ref[...] + be_ref[...])
    w = jnp.dot(a2, w3_ref[...], preferred_element_type=F32) + b3_ref[...]
    o_ref[...] = w * sc_ref[...]


def _pass3_call(z2, scale1, mu2, r2g, be2, W3, b3):
    blk = pl.BlockSpec((TCK, H), lambda i: (i, 0))
    mblk = pl.BlockSpec((TCK, 1), lambda i: (i, 0))
    row = pl.BlockSpec((1, 128), lambda i: (0, 0))
    return pl.pallas_call(
        _pass3_body,
        grid=(TGRID,),
        in_specs=[blk, mblk, row, row, row,
                  pl.BlockSpec((H, H), lambda i: (0, 0)), row],
        out_specs=blk,
        out_shape=jax.ShapeDtypeStruct((EP, H), F32),
    )(z2, scale1, mu2, r2g, be2, W3, b3)


def _update_body(h_ref, a0_ref, a1_ref, d0_ref, d1_ref, lw_ref, g_ref, b_ref,
                 o_ref):
    zn = (jnp.dot(h_ref[...], lw_ref[...], preferred_element_type=F32)
          + (a0_ref[...] + a1_ref[...]) / (d0_ref[...] + d1_ref[...]))
    mu = jnp.mean(zn, axis=0, keepdims=True)
    var = jnp.mean(jnp.square(zn - mu), axis=0, keepdims=True)
    o_ref[...] = _elu((zn - mu) * lax.rsqrt(var + 1e-5) * g_ref[...] + b_ref[...])


def _update_call(h, agg0, agg1, d0, d1, lw, g, b):
    return pl.pallas_call(
        _update_body,
        out_shape=jax.ShapeDtypeStruct((N, H), F32),
    )(h, agg0, agg1, d0, d1, lw, g, b)


def _final_body(h_ref, w_ref, b_ref, o_ref):
    o_ref[...] = jnp.dot(h_ref[...], w_ref[...], preferred_element_type=F32) + b_ref[...]


def _final_call(h, wpad, bpad):
    return pl.pallas_call(
        _final_body,
        out_shape=jax.ShapeDtypeStruct((N, 128), F32),
    )(h, wpad, bpad)


def kernel(x, edge_index, pos, y, lift_W, lift_b, lin_W, k1_W, k1_b, k2_W,
           k2_b, k3_W, k3_b, kbn1_g, kbn1_b, kbn2_g, kbn2_b, bn_g, bn_b,
           lower_W, lower_b):
    L = lin_W.shape[0]
    keys = edge_index[0] * N + edge_index[1]
    skeys2d = jnp.sort(keys).reshape(E_ROWS, 128)

    src2d, dst2d, msk2d, cnt11 = _edge_call(skeys2d)
    src3 = src2d.reshape(NW, NCH, CK)
    dst3 = dst2d.reshape(NW, NCH, CK)
    srcM = src2d.reshape(EP)
    dstM = dst2d.reshape(EP)
    msk3 = msk2d.reshape(NW, NCH, CK)
    msk1 = msk2d.reshape(EP, 1)
    cnt = cnt11[0, 0]

    degt = _deg_call(src3, msk3)
    d0 = degt[0, :N]
    d1 = degt[1, :N]

    pos128 = jnp.pad(pos, ((0, 0), (0, 128 - DIM)))
    y128 = jnp.pad(y, (0, 128 - DIM)).reshape(1, 128)
    pad_w = lambda w: jnp.pad(w, ((0, 128 - w.shape[0]), (0, 0)))
    Wx = lift_W[:F]
    Wp = pad_w(lift_W[F:F + DIM])
    Wy = pad_w(lift_W[F + DIM:F + 2 * DIM])
    lb = lift_b.reshape(1, 128)

    h = _node_call(x, pos128, y128, Wx, Wp, Wy, lb)

    pad_w3 = lambda w: jnp.pad(w, ((0, 0), (0, 128 - w.shape[1]), (0, 0)))
    Wps = pad_w3(k1_W[:, 0:DIM, :])
    Wpd = pad_w3(k1_W[:, DIM:2 * DIM, :])
    Wxs = k1_W[:, 2 * DIM:2 * DIM + F, :]
    Wxd = k1_W[:, 2 * DIM + F:2 * DIM + 2 * F, :]
    A, B = _ab_call(x, pos128, Wxs, Wps, Wxd, Wpd, L)
    z1s = [_gather_call(A[l], B[l], src3, dst3) for l in range(L)]

    for l in range(L):
        z1 = z1s[l]
        st1 = _stats1_call(z1, msk1)
        mu1 = (st1[0:1] / cnt)
        var1 = st1[1:2] / cnt - mu1 * mu1
        r1g = lax.rsqrt(var1 + 1e-5) * kbn1_g[l].reshape(1, 128)
        be1 = kbn1_b[l].reshape(1, 128)
        z2, st2 = _pass2_call(z1, msk1, mu1, r1g, be1, k2_W[l])
        mu2 = st2[0:1] / cnt
        var2 = st2[1:2] / cnt - mu2 * mu2
        r2g = lax.rsqrt(var2 + 1e-5) * kbn2_g[l].reshape(1, 128)
        be2 = kbn2_b[l].reshape(1, 128)
        w = _pass3_call(z2, msk1, mu2, r2g, be2, k3_W[l],
                        k3_b[l].reshape(1, 128))
        agg2 = _msg_call(w, h, srcM, dstM)
        h = _update_call(h, agg2[0, :N], agg2[1, :N], d0, d1, lin_W[l],
                         bn_g[l].reshape(1, 128), bn_b[l].reshape(1, 128))

    wpad = jnp.pad(lower_W, ((0, 0), (0, 128 - lower_W.shape[1])))
    bpad = jnp.pad(lower_b, (0, 128 - lower_b.shape[0])).reshape(1, 128)
    out128 = _final_call(h, wpad, bpad)
    return out128[:, :lower_W.shape[1]]
